# baseline (speedup 1.0000x reference)
import jax
import jax.numpy as jnp
from jax import lax
from jax.experimental import pallas as pl
from jax.experimental.pallas import tpu as pltpu
from jax.experimental.pallas import tpu_sc as plsc

N = 50000
D = 256
G = 1024
GH = 2 * D

ABLK = 400
NAB = 125

SEG_PER_TILE = G // 32
CHUNK = 200


def _stage_a_body(seg_ref, hv_ref, wg_ref, bg_ref, w_ref, g_ref, bnd_ref,
                  cnt_ref):
    i = pl.program_id(0)

    @pl.when(i == 0)
    def _init():
        cnt_ref[...] = jnp.zeros_like(cnt_ref)

    hv = hv_ref[...]
    wg = wg_ref[...]
    logits = jnp.sum(hv * wg, axis=1, keepdims=True) + bg_ref[0, 0]
    gate = 1.0 / (1.0 + jnp.exp(-logits))
    w_ref[...] = gate * hv
    g_ref[...] = jnp.broadcast_to(gate, (ABLK, 16))

    seg = seg_ref[0]
    thr = lax.broadcasted_iota(jnp.int32, (1, 128), 1) * SEG_PER_TILE
    below = (seg < thr).astype(jnp.float32)
    cnt_ref[...] += jnp.sum(below, axis=0, keepdims=True)

    @pl.when(i == NAB - 1)
    def _fin():
        bnd_ref[...] = cnt_ref[...]


def _sc_body(w_hbm, g_hbm, bnd_hbm, pw_hbm, pc_hbm,
             wbuf, gbuf, bndv, acc, cacc):
    cid = lax.axis_index("c")
    sid = lax.axis_index("s")
    t = sid * 2 + cid
    iota16 = lax.iota(jnp.int32, 16)

    def zs(s, c0):
        for k in range(16):
            acc[s, pl.ds(k * 16, 16)] = jnp.zeros((16,), jnp.float32)
        cacc[s, :] = jnp.zeros((16,), jnp.float32)
        return c0

    lax.fori_loop(0, SEG_PER_TILE, zs, 0)

    pltpu.sync_copy(bnd_hbm, bndv)

    def extract(idx):
        tot = jnp.float32(0.0)
        for j in range(3):
            v = bndv[0, pl.ds(j * 16, 16)]
            tot += jnp.sum(jnp.where(iota16 + j * 16 == idx, v, 0.0))
        return tot.astype(jnp.int32)

    b = [extract(t * SEG_PER_TILE + s) for s in range(SEG_PER_TILE)]
    b.append(extract(t * SEG_PER_TILE + SEG_PER_TILE))
    lo, hi = b[0], b[SEG_PER_TILE]

    def chunk(q, carry):
        off = q * CHUNK
        pltpu.sync_copy(w_hbm.at[pl.ds(off, CHUNK)], wbuf)
        pltpu.sync_copy(g_hbm.at[pl.ds(off, CHUNK)], gbuf)

        for s in range(SEG_PER_TILE):
            jlo = jnp.clip(b[s] - off, 0, CHUNK)
            jhi = jnp.clip(b[s + 1] - off, 0, CHUNK)

            @pl.when(jhi > jlo)
            def _seg(s=s, jlo=jlo, jhi=jhi):
                def row(jj, regs):
                    gsum = regs[16] + gbuf[jj]
                    return tuple(
                        regs[k] + wbuf[jj, pl.ds(k * 16, 16)]
                        for k in range(16)) + (gsum,)

                zero = jnp.zeros((16,), jnp.float32)
                regs = lax.fori_loop(jlo, jhi, row, (zero,) * 17)
                for k in range(16):
                    plsc.addupdate(acc.at[s, pl.ds(k * 16, 16)], regs[k])
                plsc.addupdate(cacc.at[s, :], regs[16])

        return carry

    lax.fori_loop(lo // CHUNK, (hi + CHUNK - 1) // CHUNK, chunk, 0)

    pltpu.sync_copy(acc, pw_hbm.at[pl.ds(t * SEG_PER_TILE, SEG_PER_TILE)])
    pltpu.sync_copy(cacc, pc_hbm.at[pl.ds(t * SEG_PER_TILE, SEG_PER_TILE)])


def _final_body(pw_ref, pc_ref, wp_ref, bp_ref, out_ref):
    s = pw_ref[...]
    c = pc_ref[:, 0:1]
    out_ref[...] = lax.dot_general(
        s, wp_ref[...], (((1,), (1,)), ((), ())),
        preferred_element_type=jnp.float32) + c * bp_ref[...]


def kernel(hv, segment_ids, W_gate, b_gate, W_proj, b_proj):
    bg = b_gate.reshape(1, 1)
    bp = b_proj.reshape(1, GH)
    seg3 = segment_ids.astype(jnp.int32).reshape(NAB, ABLK, 1)

    w, g16, bnd = pl.pallas_call(
        _stage_a_body,
        grid=(NAB,),
        in_specs=[
            pl.BlockSpec((1, ABLK, 1), lambda i: (i, 0, 0)),
            pl.BlockSpec((ABLK, D), lambda i: (i, 0)),
            pl.BlockSpec((1, D), lambda i: (0, 0)),
            pl.BlockSpec((1, 1), lambda i: (0, 0)),
        ],
        out_specs=[
            pl.BlockSpec((ABLK, D), lambda i: (i, 0)),
            pl.BlockSpec((ABLK, 16), lambda i: (i, 0)),
            pl.BlockSpec((1, 128), lambda i: (0, 0)),
        ],
        out_shape=[
            jax.ShapeDtypeStruct((N, D), jnp.float32),
            jax.ShapeDtypeStruct((N, 16), jnp.float32),
            jax.ShapeDtypeStruct((1, 128), jnp.float32),
        ],
        scratch_shapes=[pltpu.VMEM((1, 128), jnp.float32)],
    )(seg3, hv, W_gate, bg)

    mesh = plsc.VectorSubcoreMesh(core_axis_name="c", subcore_axis_name="s")
    pw, pc = pl.kernel(
        _sc_body,
        out_type=[
            jax.ShapeDtypeStruct((G, D), jnp.float32),
            jax.ShapeDtypeStruct((G, 16), jnp.float32),
        ],
        mesh=mesh,
        compiler_params=pltpu.CompilerParams(needs_layout_passes=False),
        scratch_types=[
            pltpu.VMEM((CHUNK, D), jnp.float32),
            pltpu.VMEM((CHUNK, 16), jnp.float32),
            pltpu.VMEM((1, 128), jnp.float32),
            pltpu.VMEM((SEG_PER_TILE, D), jnp.float32),
            pltpu.VMEM((SEG_PER_TILE, 16), jnp.float32),
        ],
    )(w, g16, bnd, )

    out = pl.pallas_call(
        _final_body,
        grid=(1,),
        in_specs=[
            pl.BlockSpec((G, D), lambda i: (0, 0)),
            pl.BlockSpec((G, 16), lambda i: (0, 0)),
            pl.BlockSpec((GH, D), lambda i: (0, 0)),
            pl.BlockSpec((1, GH), lambda i: (0, 0)),
        ],
        out_specs=pl.BlockSpec((G, GH), lambda i: (0, 0)),
        out_shape=jax.ShapeDtypeStruct((G, GH), jnp.float32),
    )(pw, pc, W_proj, bp)
    return out


# speedup vs baseline: 6.3300x; 6.3300x over previous
import jax
import jax.numpy as jnp
from jax import lax
from jax.experimental import pallas as pl
from jax.experimental.pallas import tpu as pltpu
from jax.experimental.pallas import tpu_sc as plsc

N = 50000
D = 256
G = 1024
GH = 2 * D

ABLK = 400
NAB = 125

SEG_PER_TILE = G // 32
CHUNK = 200


def _stage_a_body(hv_ref, wg_ref, bg_ref, w_ref, g_ref):
    hv = hv_ref[...]
    wg = wg_ref[...]
    logits = jnp.sum(hv * wg, axis=1, keepdims=True) + bg_ref[0, 0]
    gate = 1.0 / (1.0 + jnp.exp(-logits))
    w_ref[...] = gate * hv
    g_ref[...] = jnp.broadcast_to(gate, (ABLK, 16))


def _sc_body(w_hbm, g_hbm, bnd_hbm, pw_hbm, pc_hbm,
             wbuf, gbuf, bndv, acc, cacc):
    cid = lax.axis_index("c")
    sid = lax.axis_index("s")
    t = sid * 2 + cid
    iota16 = lax.iota(jnp.int32, 16)

    def zs(s, c0):
        for k in range(16):
            acc[s, pl.ds(k * 16, 16)] = jnp.zeros((16,), jnp.float32)
        cacc[s, :] = jnp.zeros((16,), jnp.float32)
        return c0

    lax.fori_loop(0, SEG_PER_TILE, zs, 0)

    pltpu.sync_copy(bnd_hbm.at[pl.ds(t * SEG_PER_TILE, 48)], bndv)

    def extract(idx):
        tot = jnp.float32(0.0)
        for j in range(3):
            v = bndv[pl.ds(j * 16, 16)]
            tot += jnp.sum(jnp.where(iota16 + j * 16 == idx, v, 0.0))
        return tot.astype(jnp.int32)

    b = [extract(s) for s in range(SEG_PER_TILE + 1)]
    lo, hi = b[0], b[SEG_PER_TILE]

    def chunk(q, carry):
        off = q * CHUNK
        pltpu.sync_copy(w_hbm.at[pl.ds(off, CHUNK)], wbuf)
        pltpu.sync_copy(g_hbm.at[pl.ds(off, CHUNK)], gbuf)

        for s in range(SEG_PER_TILE):
            jlo = jnp.clip(b[s] - off, 0, CHUNK)
            jhi = jnp.clip(b[s + 1] - off, 0, CHUNK)

            @pl.when(jhi > jlo)
            def _seg(s=s, jlo=jlo, jhi=jhi):
                def row(jj, regs):
                    gsum = regs[16] + gbuf[jj]
                    return tuple(
                        regs[k] + wbuf[jj, pl.ds(k * 16, 16)]
                        for k in range(16)) + (gsum,)

                zero = jnp.zeros((16,), jnp.float32)
                regs = lax.fori_loop(jlo, jhi, row, (zero,) * 17)
                for k in range(16):
                    plsc.addupdate(acc.at[s, pl.ds(k * 16, 16)], regs[k])
                plsc.addupdate(cacc.at[s, :], regs[16])

        return carry

    lax.fori_loop(lo // CHUNK, (hi + CHUNK - 1) // CHUNK, chunk, 0)

    pltpu.sync_copy(acc, pw_hbm.at[pl.ds(t * SEG_PER_TILE, SEG_PER_TILE)])
    pltpu.sync_copy(cacc, pc_hbm.at[pl.ds(t * SEG_PER_TILE, SEG_PER_TILE)])


def _final_body(pw_ref, pc_ref, wp_ref, bp_ref, out_ref):
    s = pw_ref[...]
    c = pc_ref[:, 0:1]
    out_ref[...] = lax.dot_general(
        s, wp_ref[...], (((1,), (1,)), ((), ())),
        preferred_element_type=jnp.float32) + c * bp_ref[...]


def kernel(hv, segment_ids, W_gate, b_gate, W_proj, b_proj):
    bg = b_gate.reshape(1, 1)
    bp = b_proj.reshape(1, GH)
    ids = segment_ids.astype(jnp.int32)
    bnd = jnp.concatenate([
        jnp.searchsorted(ids, jnp.arange(1025, dtype=jnp.int32)),
        jnp.full((15,), N, jnp.int32)]).astype(jnp.float32)

    w, g16 = pl.pallas_call(
        _stage_a_body,
        grid=(NAB,),
        in_specs=[
            pl.BlockSpec((ABLK, D), lambda i: (i, 0)),
            pl.BlockSpec((1, D), lambda i: (0, 0)),
            pl.BlockSpec((1, 1), lambda i: (0, 0)),
        ],
        out_specs=[
            pl.BlockSpec((ABLK, D), lambda i: (i, 0)),
            pl.BlockSpec((ABLK, 16), lambda i: (i, 0)),
        ],
        out_shape=[
            jax.ShapeDtypeStruct((N, D), jnp.float32),
            jax.ShapeDtypeStruct((N, 16), jnp.float32),
        ],
    )(hv, W_gate, bg)

    mesh = plsc.VectorSubcoreMesh(core_axis_name="c", subcore_axis_name="s")
    pw, pc = pl.kernel(
        _sc_body,
        out_type=[
            jax.ShapeDtypeStruct((G, D), jnp.float32),
            jax.ShapeDtypeStruct((G, 16), jnp.float32),
        ],
        mesh=mesh,
        compiler_params=pltpu.CompilerParams(needs_layout_passes=False),
        scratch_types=[
            pltpu.VMEM((CHUNK, D), jnp.float32),
            pltpu.VMEM((CHUNK, 16), jnp.float32),
            pltpu.VMEM((48,), jnp.float32),
            pltpu.VMEM((SEG_PER_TILE, D), jnp.float32),
            pltpu.VMEM((SEG_PER_TILE, 16), jnp.float32),
        ],
    )(w, g16, bnd)

    out = pl.pallas_call(
        _final_body,
        grid=(1,),
        in_specs=[
            pl.BlockSpec((G, D), lambda i: (0, 0)),
            pl.BlockSpec((G, 16), lambda i: (0, 0)),
            pl.BlockSpec((GH, D), lambda i: (0, 0)),
            pl.BlockSpec((1, GH), lambda i: (0, 0)),
        ],
        out_specs=pl.BlockSpec((G, GH), lambda i: (0, 0)),
        out_shape=jax.ShapeDtypeStruct((G, GH), jnp.float32),
    )(pw, pc, W_proj, bp)
    return out
